# baseline (speedup 1.0000x reference)
import functools

import jax
import jax.numpy as jnp
from jax import lax
from jax.experimental import pallas as pl
from jax.experimental.pallas import tpu as pltpu
from jax.experimental.pallas import tpu_sc as plsc

ROWS = 16384
COLS = 2048
TOTAL = 12 * (16 + 32 + 64 + 128)
LANES = 16
N_CHUNK = TOTAL // LANES
NW = 32
R = 8


@jax.jit
def _sc_gather(input_tensor, col_idx):
    rows_per_w = ROWS // NW
    nblocks = rows_per_w // R
    mesh = plsc.VectorSubcoreMesh(core_axis_name="c", subcore_axis_name="s")

    @functools.partial(
        pl.kernel,
        out_type=jax.ShapeDtypeStruct((ROWS, TOTAL), jnp.float32),
        mesh=mesh,
        compiler_params=pltpu.CompilerParams(needs_layout_passes=False),
        scratch_types=[
            pltpu.VMEM((TOTAL,), jnp.int32),
            pltpu.VMEM((R * COLS,), jnp.float32),
            pltpu.VMEM((R, TOTAL), jnp.float32),
        ],
    )
    def k(in_hbm, idx_hbm, out_hbm, idx_v, in_v, out_v):
        wid = lax.axis_index("s") * 2 + lax.axis_index("c")
        base = wid * rows_per_w
        pltpu.sync_copy(idx_hbm, idx_v)

        @pl.loop(0, nblocks)
        def _block(b):
            row0 = base + b * R
            pltpu.sync_copy(in_hbm.at[pl.ds(row0 * COLS, R * COLS)], in_v)

            @pl.loop(0, N_CHUNK)
            def _chunk(c):
                cv = idx_v[pl.ds(c * LANES, LANES)]
                for r in range(R):
                    out_v[r, pl.ds(c * LANES, LANES)] = plsc.load_gather(
                        in_v, [cv + (r * COLS)]
                    )

            pltpu.sync_copy(out_v, out_hbm.at[pl.ds(row0, R)])

    return k(input_tensor.reshape(ROWS * COLS), col_idx)


def kernel(input_tensor, slices):
    starts = slices[:, 0].astype(jnp.int32)
    ends = slices[:, 1].astype(jnp.int32)
    lengths = ends - starts
    cum_ends = jnp.cumsum(lengths)
    cum_starts = cum_ends - lengths
    j = jnp.arange(TOTAL, dtype=jnp.int32)
    seg = jnp.searchsorted(cum_ends, j, side="right")
    col_idx = (starts[seg] + (j - cum_starts[seg])).astype(jnp.int32)
    return _sc_gather(input_tensor, col_idx)


# speedup vs baseline: 1.1402x; 1.1402x over previous
import functools

import jax
import jax.numpy as jnp
from jax import lax
from jax.experimental import pallas as pl
from jax.experimental.pallas import tpu as pltpu
from jax.experimental.pallas import tpu_sc as plsc

ROWS = 16384
COLS = 2048
TOTAL = 12 * (16 + 32 + 64 + 128)
LANES = 16
N_CHUNK = TOTAL // LANES
NW = 32
RPW = ROWS // NW
R = 8
NB = RPW // R


@jax.jit
def _sc_gather(input_tensor, col_idx):
    mesh = plsc.VectorSubcoreMesh(core_axis_name="c", subcore_axis_name="s")

    @functools.partial(
        pl.kernel,
        out_type=jax.ShapeDtypeStruct((ROWS, TOTAL), jnp.float32),
        mesh=mesh,
        compiler_params=pltpu.CompilerParams(needs_layout_passes=False),
        scratch_types=[
            pltpu.VMEM((TOTAL,), jnp.int32),
            pltpu.VMEM((R * COLS,), jnp.float32),
            pltpu.VMEM((R * COLS,), jnp.float32),
            pltpu.VMEM((R, TOTAL), jnp.float32),
            pltpu.VMEM((R, TOTAL), jnp.float32),
            pltpu.SemaphoreType.DMA((2,)),
            pltpu.SemaphoreType.DMA((2,)),
        ],
    )
    def k(in_hbm, idx_hbm, out_hbm, idx_v, in_v0, in_v1, out_v0, out_v1,
          sin, sout):
        in_bufs = (in_v0, in_v1)
        out_bufs = (out_v0, out_v1)
        wid = lax.axis_index("s") * 2 + lax.axis_index("c")
        row0 = wid * RPW
        pltpu.sync_copy(idx_hbm, idx_v)
        pltpu.async_copy(
            in_hbm.at[pl.ds(row0 * COLS, R * COLS)], in_v0, sin.at[0]
        )

        @pl.loop(0, NB // 2)
        def _pair(p):
            for phase in (0, 1):
                b = 2 * p + phase
                buf = phase
                nbuf = 1 - phase
                pltpu.make_async_copy(
                    in_hbm.at[pl.ds(0, R * COLS)], in_bufs[buf], sin.at[buf]
                ).wait()

                @pl.when(b + 1 < NB)
                def _():
                    pltpu.async_copy(
                        in_hbm.at[pl.ds((row0 + (b + 1) * R) * COLS, R * COLS)],
                        in_bufs[nbuf],
                        sin.at[nbuf],
                    )

                @pl.when(b >= 2)
                def _():
                    pltpu.make_async_copy(
                        out_bufs[buf], out_hbm.at[pl.ds(0, R)], sout.at[buf]
                    ).wait()

                @pl.loop(0, N_CHUNK)
                def _chunk(c):
                    cv = idx_v[pl.ds(c * LANES, LANES)]
                    for r in range(R):
                        out_bufs[buf][r, pl.ds(c * LANES, LANES)] = (
                            plsc.load_gather(in_bufs[buf], [cv + (r * COLS)])
                        )

                pltpu.async_copy(
                    out_bufs[buf],
                    out_hbm.at[pl.ds(row0 + b * R, R)],
                    sout.at[buf],
                )

        for buf in (0, 1):
            pltpu.make_async_copy(
                out_bufs[buf], out_hbm.at[pl.ds(0, R)], sout.at[buf]
            ).wait()

    return k(input_tensor.reshape(ROWS * COLS), col_idx)


def kernel(input_tensor, slices):
    starts = slices[:, 0].astype(jnp.int32)
    ends = slices[:, 1].astype(jnp.int32)
    lengths = ends - starts
    cum_ends = jnp.cumsum(lengths)
    cum_starts = cum_ends - lengths
    j = jnp.arange(TOTAL, dtype=jnp.int32)
    seg = jnp.searchsorted(cum_ends, j, side="right")
    col_idx = (starts[seg] + (j - cum_starts[seg])).astype(jnp.int32)
    return _sc_gather(input_tensor, col_idx)


# speedup vs baseline: 1.1411x; 1.0008x over previous
import functools

import jax
import jax.numpy as jnp
from jax import lax
from jax.experimental import pallas as pl
from jax.experimental.pallas import tpu as pltpu
from jax.experimental.pallas import tpu_sc as plsc

ROWS = 16384
COLS = 2048
TOTAL = 12 * (16 + 32 + 64 + 128)
LANES = 16
N_CHUNK = TOTAL // LANES
NW = 32
RPW = ROWS // NW
R = 8
NB = RPW // R


@jax.jit
def _sc_gather(input_tensor, col_idx):
    mesh = plsc.VectorSubcoreMesh(core_axis_name="c", subcore_axis_name="s")

    @functools.partial(
        pl.kernel,
        out_type=jax.ShapeDtypeStruct((ROWS, TOTAL), jnp.float32),
        mesh=mesh,
        compiler_params=pltpu.CompilerParams(needs_layout_passes=False),
        scratch_types=[
            pltpu.VMEM((TOTAL,), jnp.int32),
            pltpu.VMEM((R, COLS), jnp.float32),
            pltpu.VMEM((R, COLS), jnp.float32),
            pltpu.VMEM((R, TOTAL), jnp.float32),
            pltpu.VMEM((R, TOTAL), jnp.float32),
            pltpu.SemaphoreType.DMA((2,)),
            pltpu.SemaphoreType.DMA((2,)),
        ],
    )
    def k(in_hbm, idx_hbm, out_hbm, idx_v, in_v0, in_v1, out_v0, out_v1,
          sin, sout):
        in_bufs = (in_v0, in_v1)
        out_bufs = (out_v0, out_v1)
        wid = lax.axis_index("s") * 2 + lax.axis_index("c")
        row0 = wid * RPW
        pltpu.sync_copy(idx_hbm, idx_v)
        pltpu.async_copy(in_hbm.at[pl.ds(row0, R)], in_v0, sin.at[0])

        @pl.loop(0, NB // 2)
        def _pair(p):
            for phase in (0, 1):
                b = 2 * p + phase
                buf = phase
                nbuf = 1 - phase
                pltpu.make_async_copy(
                    in_hbm.at[pl.ds(0, R)], in_bufs[buf], sin.at[buf]
                ).wait()

                @pl.when(b + 1 < NB)
                def _():
                    pltpu.async_copy(
                        in_hbm.at[pl.ds(row0 + (b + 1) * R, R)],
                        in_bufs[nbuf],
                        sin.at[nbuf],
                    )

                @pl.when(b >= 2)
                def _():
                    pltpu.make_async_copy(
                        out_bufs[buf], out_hbm.at[pl.ds(0, R)], sout.at[buf]
                    ).wait()

                @pl.loop(0, N_CHUNK)
                def _chunk(c):
                    cv = idx_v[pl.ds(c * LANES, LANES)]
                    for r in range(R):
                        rv = jnp.full((LANES,), r, jnp.int32)
                        out_bufs[buf][r, pl.ds(c * LANES, LANES)] = (
                            plsc.load_gather(in_bufs[buf], [rv, cv])
                        )

                pltpu.async_copy(
                    out_bufs[buf],
                    out_hbm.at[pl.ds(row0 + b * R, R)],
                    sout.at[buf],
                )

        for buf in (0, 1):
            pltpu.make_async_copy(
                out_bufs[buf], out_hbm.at[pl.ds(0, R)], sout.at[buf]
            ).wait()

    return k(input_tensor, col_idx)


def kernel(input_tensor, slices):
    starts = slices[:, 0].astype(jnp.int32)
    ends = slices[:, 1].astype(jnp.int32)
    lengths = ends - starts
    cum_ends = jnp.cumsum(lengths)
    cum_starts = cum_ends - lengths
    j = jnp.arange(TOTAL, dtype=jnp.int32)
    seg = jnp.searchsorted(cum_ends, j, side="right")
    col_idx = (starts[seg] + (j - cum_starts[seg])).astype(jnp.int32)
    return _sc_gather(input_tensor, col_idx)


# speedup vs baseline: 1.7388x; 1.5238x over previous
import functools

import jax
import jax.numpy as jnp
from jax import lax
from jax.experimental import pallas as pl
from jax.experimental.pallas import tpu as pltpu
from jax.experimental.pallas import tpu_sc as plsc

ROWS = 16384
COLS = 2048
TOTAL = 12 * (16 + 32 + 64 + 128)
LANES = 16
N_CHUNK = TOTAL // LANES
NW = 32
RPW = ROWS // NW
R = 8
NB = RPW // R


@jax.jit
def _sc_gather(input_tensor, col_idx):
    mesh = plsc.VectorSubcoreMesh(core_axis_name="c", subcore_axis_name="s")

    @functools.partial(
        pl.kernel,
        out_type=jax.ShapeDtypeStruct((ROWS, TOTAL), jnp.float32),
        mesh=mesh,
        compiler_params=pltpu.CompilerParams(needs_layout_passes=False),
        scratch_types=[
            pltpu.VMEM((TOTAL,), jnp.int32),
            pltpu.VMEM((R, COLS), jnp.float32),
            pltpu.VMEM((R, COLS), jnp.float32),
            pltpu.VMEM((R, TOTAL), jnp.float32),
            pltpu.VMEM((R, TOTAL), jnp.float32),
            pltpu.SemaphoreType.DMA((2,)),
            pltpu.SemaphoreType.DMA((2,)),
        ],
    )
    def k(in_hbm, idx_hbm, out_hbm, idx_v, in_v0, in_v1, out_v0, out_v1,
          sin, sout):
        in_bufs = (in_v0, in_v1)
        out_bufs = (out_v0, out_v1)
        wid = lax.axis_index("s") * 2 + lax.axis_index("c")
        row0 = wid * RPW
        pltpu.sync_copy(idx_hbm, idx_v)
        pltpu.async_copy(in_hbm.at[pl.ds(row0, R)], in_v0, sin.at[0])

        @pl.loop(0, NB // 2)
        def _pair(p):
            for phase in (0, 1):
                b = 2 * p + phase
                buf = phase
                nbuf = 1 - phase
                pltpu.make_async_copy(
                    in_hbm.at[pl.ds(0, R)], in_bufs[buf], sin.at[buf]
                ).wait()

                @pl.when(b + 1 < NB)
                def _():
                    pltpu.async_copy(
                        in_hbm.at[pl.ds(row0 + (b + 1) * R, R)],
                        in_bufs[nbuf],
                        sin.at[nbuf],
                    )

                @pl.when(b >= 2)
                def _():
                    pltpu.make_async_copy(
                        out_bufs[buf], out_hbm.at[pl.ds(0, R)], sout.at[buf]
                    ).wait()

                @pl.loop(0, N_CHUNK)
                def _chunk(c):
                    cv = idx_v[pl.ds(c * LANES, LANES)]
                    for r in range(R):
                        rv = jnp.full((LANES,), r, jnp.int32)
                        out_bufs[buf][r, pl.ds(c * LANES, LANES)] = (
                            plsc.load_gather(in_bufs[buf], [rv, cv])
                        )

                pltpu.async_copy(
                    out_bufs[buf],
                    out_hbm.at[pl.ds(row0 + b * R, R)],
                    sout.at[buf],
                )

        for buf in (0, 1):
            pltpu.make_async_copy(
                out_bufs[buf], out_hbm.at[pl.ds(0, R)], sout.at[buf]
            ).wait()

    return k(input_tensor, col_idx)


def kernel(input_tensor, slices):
    starts = slices[:, 0].astype(jnp.int32)
    ends = slices[:, 1].astype(jnp.int32)
    lengths = ends - starts
    cum_ends = jnp.cumsum(lengths)
    cum_starts = cum_ends - lengths
    j = jnp.arange(TOTAL, dtype=jnp.int32)
    seg = jnp.sum(
        (cum_ends[:, None] <= j[None, :]).astype(jnp.int32), axis=0
    )
    col_idx = (starts[seg] + (j - cum_starts[seg])).astype(jnp.int32)
    return _sc_gather(input_tensor, col_idx)


# speedup vs baseline: 3.2544x; 1.8717x over previous
import functools

import jax
import jax.numpy as jnp
from jax import lax
from jax.experimental import pallas as pl
from jax.experimental.pallas import tpu as pltpu
from jax.experimental.pallas import tpu_sc as plsc

ROWS = 16384
COLS = 2048
TOTAL = 12 * (16 + 32 + 64 + 128)
LANES = 16
N_CHUNK = TOTAL // LANES
NW = 32
RPW = ROWS // NW
R = 8
NB = RPW // R


@jax.jit
def _sc_gather(input_tensor, col_idx):
    mesh = plsc.VectorSubcoreMesh(core_axis_name="c", subcore_axis_name="s")

    @functools.partial(
        pl.kernel,
        out_type=jax.ShapeDtypeStruct((ROWS, TOTAL), jnp.float32),
        mesh=mesh,
        compiler_params=pltpu.CompilerParams(needs_layout_passes=False),
        scratch_types=[
            pltpu.VMEM((TOTAL,), jnp.int32),
            pltpu.VMEM((R, COLS), jnp.float32),
            pltpu.VMEM((R, COLS), jnp.float32),
            pltpu.VMEM((R, TOTAL), jnp.float32),
            pltpu.VMEM((R, TOTAL), jnp.float32),
            pltpu.SemaphoreType.DMA((2,)),
            pltpu.SemaphoreType.DMA((2,)),
        ],
    )
    def k(in_hbm, idx_hbm, out_hbm, idx_v, in_v0, in_v1, out_v0, out_v1,
          sin, sout):
        in_bufs = (in_v0, in_v1)
        out_bufs = (out_v0, out_v1)
        wid = lax.axis_index("s") * 2 + lax.axis_index("c")
        row0 = wid * RPW
        pltpu.sync_copy(idx_hbm, idx_v)
        pltpu.async_copy(in_hbm.at[pl.ds(row0, R)], in_v0, sin.at[0])

        @pl.loop(0, NB // 2)
        def _pair(p):
            for phase in (0, 1):
                b = 2 * p + phase
                buf = phase
                nbuf = 1 - phase
                pltpu.make_async_copy(
                    in_hbm.at[pl.ds(0, R)], in_bufs[buf], sin.at[buf]
                ).wait()

                @pl.when(b + 1 < NB)
                def _():
                    pltpu.async_copy(
                        in_hbm.at[pl.ds(row0 + (b + 1) * R, R)],
                        in_bufs[nbuf],
                        sin.at[nbuf],
                    )

                @pl.when(b >= 2)
                def _():
                    pltpu.make_async_copy(
                        out_bufs[buf], out_hbm.at[pl.ds(0, R)], sout.at[buf]
                    ).wait()

                @plsc.parallel_loop(0, N_CHUNK, unroll=4)
                def _chunk(c):
                    cv = idx_v[pl.ds(c * LANES, LANES)]
                    for r in range(R):
                        rv = jnp.full((LANES,), r, jnp.int32)
                        out_bufs[buf][r, pl.ds(c * LANES, LANES)] = (
                            plsc.load_gather(in_bufs[buf], [rv, cv])
                        )

                pltpu.async_copy(
                    out_bufs[buf],
                    out_hbm.at[pl.ds(row0 + b * R, R)],
                    sout.at[buf],
                )

        for buf in (0, 1):
            pltpu.make_async_copy(
                out_bufs[buf], out_hbm.at[pl.ds(0, R)], sout.at[buf]
            ).wait()

    return k(input_tensor, col_idx)


def kernel(input_tensor, slices):
    starts = slices[:, 0].astype(jnp.int32)
    ends = slices[:, 1].astype(jnp.int32)
    lengths = ends - starts
    cum_ends = jnp.cumsum(lengths)
    cum_starts = cum_ends - lengths
    j = jnp.arange(TOTAL, dtype=jnp.int32)
    seg = jnp.sum(
        (cum_ends[:, None] <= j[None, :]).astype(jnp.int32), axis=0
    )
    col_idx = (starts[seg] + (j - cum_starts[seg])).astype(jnp.int32)
    return _sc_gather(input_tensor, col_idx)


# speedup vs baseline: 3.9678x; 1.2192x over previous
import functools

import jax
import jax.numpy as jnp
from jax import lax
from jax.experimental import pallas as pl
from jax.experimental.pallas import tpu as pltpu
from jax.experimental.pallas import tpu_sc as plsc

ROWS = 16384
COLS = 2048
TOTAL = 12 * (16 + 32 + 64 + 128)
LANES = 16
N_CHUNK = TOTAL // LANES
NW = 32
RPW = ROWS // NW
R = 8
NB = RPW // R


@jax.jit
def _sc_gather(input_tensor, col_idx):
    mesh = plsc.VectorSubcoreMesh(core_axis_name="c", subcore_axis_name="s")

    @functools.partial(
        pl.kernel,
        out_type=jax.ShapeDtypeStruct((ROWS, TOTAL), jnp.float32),
        mesh=mesh,
        compiler_params=pltpu.CompilerParams(needs_layout_passes=False),
        scratch_types=[
            pltpu.VMEM((TOTAL,), jnp.int32),
            pltpu.VMEM((R, COLS), jnp.float32),
            pltpu.VMEM((R, COLS), jnp.float32),
            pltpu.VMEM((R, TOTAL), jnp.float32),
            pltpu.VMEM((R, TOTAL), jnp.float32),
            pltpu.SemaphoreType.DMA((2,)),
            pltpu.SemaphoreType.DMA((2,)),
        ],
    )
    def k(in_hbm, idx_hbm, out_hbm, idx_v, in_v0, in_v1, out_v0, out_v1,
          sin, sout):
        in_bufs = (in_v0, in_v1)
        out_bufs = (out_v0, out_v1)
        wid = lax.axis_index("s") * 2 + lax.axis_index("c")
        row0 = wid * RPW
        pltpu.sync_copy(idx_hbm, idx_v)
        pltpu.async_copy(in_hbm.at[pl.ds(row0, R)], in_v0, sin.at[0])

        @pl.loop(0, NB // 2)
        def _pair(p):
            for phase in (0, 1):
                b = 2 * p + phase
                buf = phase
                nbuf = 1 - phase
                pltpu.make_async_copy(
                    in_hbm.at[pl.ds(0, R)], in_bufs[buf], sin.at[buf]
                ).wait()

                @pl.when(b + 1 < NB)
                def _():
                    pltpu.async_copy(
                        in_hbm.at[pl.ds(row0 + (b + 1) * R, R)],
                        in_bufs[nbuf],
                        sin.at[nbuf],
                    )

                @pl.when(b >= 2)
                def _():
                    pltpu.make_async_copy(
                        out_bufs[buf], out_hbm.at[pl.ds(0, R)], sout.at[buf]
                    ).wait()

                @plsc.parallel_loop(0, N_CHUNK, unroll=4)
                def _chunk(c):
                    cv = idx_v[pl.ds(c * LANES, LANES)]
                    for r in range(R):
                        rv = jnp.full((LANES,), r, jnp.int32)
                        out_bufs[buf][r, pl.ds(c * LANES, LANES)] = (
                            plsc.load_gather(in_bufs[buf], [rv, cv])
                        )

                pltpu.async_copy(
                    out_bufs[buf],
                    out_hbm.at[pl.ds(row0 + b * R, R)],
                    sout.at[buf],
                )

        for buf in (0, 1):
            pltpu.make_async_copy(
                out_bufs[buf], out_hbm.at[pl.ds(0, R)], sout.at[buf]
            ).wait()

    return k(input_tensor, col_idx)


def kernel(input_tensor, slices):
    starts = slices[:, 0].astype(jnp.int32)
    ends = slices[:, 1].astype(jnp.int32)
    lengths = ends - starts
    cum_ends = jnp.cumsum(lengths)
    cum_starts = cum_ends - lengths
    j = jnp.arange(TOTAL, dtype=jnp.int32)
    delta = starts - cum_starts
    ddelta = delta[1:] - delta[:-1]
    steps = jnp.where(
        cum_ends[:-1, None] <= j[None, :], ddelta[:, None], 0
    )
    col_idx = j + delta[0] + jnp.sum(steps, axis=0, dtype=jnp.int32)
    return _sc_gather(input_tensor, col_idx)


# speedup vs baseline: 3.9712x; 1.0009x over previous
import functools

import jax
import jax.numpy as jnp
from jax import lax
from jax.experimental import pallas as pl
from jax.experimental.pallas import tpu as pltpu
from jax.experimental.pallas import tpu_sc as plsc

ROWS = 16384
COLS = 2048
TOTAL = 12 * (16 + 32 + 64 + 128)
LANES = 16
N_CHUNK = TOTAL // LANES
NW = 32
RPW = ROWS // NW
R = 8
NB = RPW // R


@jax.jit
def _sc_gather(input_tensor, col_idx):
    mesh = plsc.VectorSubcoreMesh(core_axis_name="c", subcore_axis_name="s")

    @functools.partial(
        pl.kernel,
        out_type=jax.ShapeDtypeStruct((ROWS, TOTAL), jnp.float32),
        mesh=mesh,
        compiler_params=pltpu.CompilerParams(needs_layout_passes=False),
        scratch_types=[
            pltpu.VMEM((TOTAL,), jnp.int32),
            pltpu.VMEM((R, COLS), jnp.float32),
            pltpu.VMEM((R, COLS), jnp.float32),
            pltpu.VMEM((R, TOTAL), jnp.float32),
            pltpu.VMEM((R, TOTAL), jnp.float32),
            pltpu.SemaphoreType.DMA((2,)),
            pltpu.SemaphoreType.DMA((2,)),
        ],
    )
    def k(in_hbm, idx_hbm, out_hbm, idx_v, in_v0, in_v1, out_v0, out_v1,
          sin, sout):
        in_bufs = (in_v0, in_v1)
        out_bufs = (out_v0, out_v1)
        wid = lax.axis_index("s") * 2 + lax.axis_index("c")
        row0 = wid * RPW
        pltpu.sync_copy(idx_hbm, idx_v)
        pltpu.async_copy(in_hbm.at[pl.ds(row0, R)], in_v0, sin.at[0])

        @pl.loop(0, NB // 2)
        def _pair(p):
            for phase in (0, 1):
                b = 2 * p + phase
                buf = phase
                nbuf = 1 - phase
                pltpu.make_async_copy(
                    in_hbm.at[pl.ds(0, R)], in_bufs[buf], sin.at[buf]
                ).wait()

                @pl.when(b + 1 < NB)
                def _():
                    pltpu.async_copy(
                        in_hbm.at[pl.ds(row0 + (b + 1) * R, R)],
                        in_bufs[nbuf],
                        sin.at[nbuf],
                    )

                @pl.when(b >= 2)
                def _():
                    pltpu.make_async_copy(
                        out_bufs[buf], out_hbm.at[pl.ds(0, R)], sout.at[buf]
                    ).wait()

                @plsc.parallel_loop(0, N_CHUNK, unroll=8)
                def _chunk(c):
                    cv = idx_v[pl.ds(c * LANES, LANES)]
                    for r in range(R):
                        rv = jnp.full((LANES,), r, jnp.int32)
                        out_bufs[buf][r, pl.ds(c * LANES, LANES)] = (
                            plsc.load_gather(in_bufs[buf], [rv, cv])
                        )

                pltpu.async_copy(
                    out_bufs[buf],
                    out_hbm.at[pl.ds(row0 + b * R, R)],
                    sout.at[buf],
                )

        for buf in (0, 1):
            pltpu.make_async_copy(
                out_bufs[buf], out_hbm.at[pl.ds(0, R)], sout.at[buf]
            ).wait()

    return k(input_tensor, col_idx)


def kernel(input_tensor, slices):
    starts = slices[:, 0].astype(jnp.int32)
    ends = slices[:, 1].astype(jnp.int32)
    lengths = ends - starts
    cum_ends = jnp.cumsum(lengths)
    cum_starts = cum_ends - lengths
    j = jnp.arange(TOTAL, dtype=jnp.int32)
    delta = starts - cum_starts
    ddelta = delta[1:] - delta[:-1]
    steps = jnp.where(
        cum_ends[:-1, None] <= j[None, :], ddelta[:, None], 0
    )
    col_idx = j + delta[0] + jnp.sum(steps, axis=0, dtype=jnp.int32)
    return _sc_gather(input_tensor, col_idx)


# speedup vs baseline: 4.0215x; 1.0127x over previous
import functools

import jax
import jax.numpy as jnp
from jax import lax
from jax.experimental import pallas as pl
from jax.experimental.pallas import tpu as pltpu
from jax.experimental.pallas import tpu_sc as plsc

ROWS = 16384
COLS = 2048
TOTAL = 12 * (16 + 32 + 64 + 128)
LANES = 16
N_CHUNK = TOTAL // LANES
NW = 32
RPW = ROWS // NW
R = 8
NB = RPW // R
DEPTH = 3


@jax.jit
def _sc_gather(input_tensor, col_idx):
    mesh = plsc.VectorSubcoreMesh(core_axis_name="c", subcore_axis_name="s")

    @functools.partial(
        pl.kernel,
        out_type=jax.ShapeDtypeStruct((ROWS, TOTAL), jnp.float32),
        mesh=mesh,
        compiler_params=pltpu.CompilerParams(needs_layout_passes=False),
        scratch_types=[
            pltpu.VMEM((TOTAL,), jnp.int32),
            pltpu.VMEM((R, COLS), jnp.float32),
            pltpu.VMEM((R, COLS), jnp.float32),
            pltpu.VMEM((R, COLS), jnp.float32),
            pltpu.VMEM((R, TOTAL), jnp.float32),
            pltpu.VMEM((R, TOTAL), jnp.float32),
            pltpu.VMEM((R, TOTAL), jnp.float32),
            pltpu.SemaphoreType.DMA((DEPTH,)),
            pltpu.SemaphoreType.DMA((DEPTH,)),
        ],
    )
    def k(in_hbm, idx_hbm, out_hbm, idx_v, in_v0, in_v1, in_v2,
          out_v0, out_v1, out_v2, sin, sout):
        in_bufs = (in_v0, in_v1, in_v2)
        out_bufs = (out_v0, out_v1, out_v2)
        wid = lax.axis_index("s") * 2 + lax.axis_index("c")
        row0 = wid * RPW
        pltpu.sync_copy(idx_hbm, idx_v)
        pltpu.async_copy(in_hbm.at[pl.ds(row0, R)], in_v0, sin.at[0])
        pltpu.async_copy(in_hbm.at[pl.ds(row0 + R, R)], in_v1, sin.at[1])

        def step(b, ph):
            buf = ph
            pltpu.make_async_copy(
                in_hbm.at[pl.ds(0, R)], in_bufs[buf], sin.at[buf]
            ).wait()

            @pl.when(b + 2 < NB)
            def _():
                nxt = (ph + 2) % DEPTH
                pltpu.async_copy(
                    in_hbm.at[pl.ds(row0 + (b + 2) * R, R)],
                    in_bufs[nxt],
                    sin.at[nxt],
                )

            @pl.when(b >= DEPTH)
            def _():
                pltpu.make_async_copy(
                    out_bufs[buf], out_hbm.at[pl.ds(0, R)], sout.at[buf]
                ).wait()

            @plsc.parallel_loop(0, N_CHUNK, unroll=4)
            def _chunk(c):
                cv = idx_v[pl.ds(c * LANES, LANES)]
                for r in range(R):
                    rv = jnp.full((LANES,), r, jnp.int32)
                    out_bufs[buf][r, pl.ds(c * LANES, LANES)] = (
                        plsc.load_gather(in_bufs[buf], [rv, cv])
                    )

            pltpu.async_copy(
                out_bufs[buf],
                out_hbm.at[pl.ds(row0 + b * R, R)],
                sout.at[buf],
            )

        @pl.loop(0, NB // DEPTH)
        def _round(t):
            for ph in range(DEPTH):
                step(DEPTH * t + ph, ph)

        for b in range(DEPTH * (NB // DEPTH), NB):
            step(b, b % DEPTH)

        for buf in range(DEPTH):
            pltpu.make_async_copy(
                out_bufs[buf], out_hbm.at[pl.ds(0, R)], sout.at[buf]
            ).wait()

    return k(input_tensor, col_idx)


def kernel(input_tensor, slices):
    starts = slices[:, 0].astype(jnp.int32)
    ends = slices[:, 1].astype(jnp.int32)
    lengths = ends - starts
    cum_ends = jnp.cumsum(lengths)
    cum_starts = cum_ends - lengths
    j = jnp.arange(TOTAL, dtype=jnp.int32)
    delta = starts - cum_starts
    ddelta = delta[1:] - delta[:-1]
    steps = jnp.where(
        cum_ends[:-1, None] <= j[None, :], ddelta[:, None], 0
    )
    col_idx = j + delta[0] + jnp.sum(steps, axis=0, dtype=jnp.int32)
    return _sc_gather(input_tensor, col_idx)
